# baseline (device time: 11747 ns/iter reference)
import jax
import jax.numpy as jnp
from jax import lax
from jax.experimental import pallas as pl
from jax.experimental.pallas import tpu as pltpu

N_DEV = 16
N_TOK = 256
D_IN = 128
D_OUT = 256
N_EXP = 32
EXP_PER_DEV = N_EXP // N_DEV
CAP = 6
N_SLOT = EXP_PER_DEV * CAP
TOK_PER_DEV = N_TOK // N_DEV


def kernel(x, router_W, route_idx, expert_W):
    def body(x_ref, idx_ref, w_ref, out_ref,
             meta_ref, vcol_ref, send_buf, send_sems, recv_sems):
        p = lax.axis_index("i")

        out_ref[...] = jnp.zeros_like(out_ref)

        barrier = pltpu.get_barrier_semaphore()
        for d in range(1, N_DEV):
            pl.semaphore_signal(
                barrier, inc=1,
                device_id=((p + d) % N_DEV,),
                device_id_type=pl.DeviceIdType.MESH,
            )

        v = idx_ref[...].reshape(N_TOK, 1)
        vcol_ref[...] = v
        lane_e = lax.broadcasted_iota(jnp.int32, (N_TOK, N_EXP), 1)
        one_hot = (v == lane_e).astype(jnp.float32)
        row_i = lax.broadcasted_iota(jnp.int32, (N_TOK, N_TOK), 0)
        col_i = lax.broadcasted_iota(jnp.int32, (N_TOK, N_TOK), 1)
        lower = (col_i <= row_i).astype(jnp.float32)
        counts = jnp.dot(lower.astype(jnp.bfloat16),
                         one_hot.astype(jnp.bfloat16),
                         preferred_element_type=jnp.float32)
        occ = jnp.sum(one_hot * counts, axis=1, keepdims=True)
        meta_ref[...] = occ

        s_iota = lax.broadcasted_iota(jnp.int32, (N_TOK, N_SLOT), 1)
        e_s = p * EXP_PER_DEV + s_iota // CAP
        c_s = (s_iota % CAP + 1).astype(jnp.float32)
        S = jnp.logical_and(v == e_s, occ == c_s).astype(jnp.float32)

        cdims = (((0,), (0,)), ((), ()))
        X = lax.dot_general(S, x_ref[...], cdims,
                            preferred_element_type=jnp.float32)
        row_iota = lax.broadcasted_iota(
            jnp.int32, (N_TOK, 2), 0).astype(jnp.float32)
        aux = jnp.where(
            lax.broadcasted_iota(jnp.int32, (N_TOK, 2), 1) == 0,
            row_iota, 1.0)
        R = lax.dot_general(S, aux, cdims,
                            preferred_element_type=jnp.float32)
        t_enc = R[:, :1] + 512.0 * (1.0 - jnp.minimum(R[:, 1:2], 1.0))

        X_bf = X.astype(jnp.bfloat16)
        y0 = jnp.dot(X_bf, w_ref[0].astype(jnp.bfloat16),
                     preferred_element_type=jnp.float32)
        y1 = jnp.dot(X_bf, w_ref[1].astype(jnp.bfloat16),
                     preferred_element_type=jnp.float32)
        first_half = lax.broadcasted_iota(jnp.int32, (N_SLOT, 1), 0) < CAP
        send_buf[...] = jnp.where(first_half, y0, y1)

        occ_mine = meta_ref[pl.ds(p * TOK_PER_DEV, TOK_PER_DEV), :]
        v_mine = vcol_ref[pl.ds(p * TOK_PER_DEV, TOK_PER_DEV), :]
        expected = jnp.logical_and(
            occ_mine <= float(CAP), v_mine // EXP_PER_DEV != p
        ).astype(jnp.int32)
        j_iota = lax.broadcasted_iota(jnp.int32, (TOK_PER_DEV, 1), 0)
        exp_bits = jnp.sum(expected << j_iota)

        slot_enc = [t_enc[s, 0].astype(jnp.int32) for s in range(N_SLOT)]
        slot_on = [t < N_TOK for t in slot_enc]
        slot_dst = [t % N_TOK // TOK_PER_DEV for t in slot_enc]
        slot_row = [t % TOK_PER_DEV for t in slot_enc]

        pl.semaphore_wait(barrier, N_DEV - 1)

        def send_rdma(s):
            return pltpu.make_async_remote_copy(
                src_ref=send_buf.at[pl.ds(s, 1), :],
                dst_ref=out_ref.at[pl.ds(slot_row[s], 1), :],
                send_sem=send_sems.at[s],
                recv_sem=recv_sems.at[slot_row[s]],
                device_id=(slot_dst[s],),
                device_id_type=pl.DeviceIdType.MESH,
            )

        for s in range(N_SLOT):
            @pl.when(jnp.logical_and(slot_on[s], slot_dst[s] == p))
            def _(s=s):
                out_ref[pl.ds(slot_row[s], 1), :] = send_buf[pl.ds(s, 1), :]

            @pl.when(jnp.logical_and(slot_on[s], slot_dst[s] != p))
            def _(s=s):
                send_rdma(s).start()

        for j in range(TOK_PER_DEV):
            @pl.when(jnp.bitwise_and(lax.shift_right_logical(exp_bits, j), 1) == 1)
            def _(j=j):
                recv = pltpu.make_async_remote_copy(
                    src_ref=send_buf.at[pl.ds(0, 1), :],
                    dst_ref=out_ref.at[pl.ds(j, 1), :],
                    send_sem=send_sems.at[0],
                    recv_sem=recv_sems.at[j],
                    device_id=(0,),
                    device_id_type=pl.DeviceIdType.MESH,
                )
                recv.wait_recv()

        for s in range(N_SLOT):
            @pl.when(jnp.logical_and(slot_on[s], slot_dst[s] != p))
            def _(s=s):
                send_rdma(s).wait_send()

    return pl.pallas_call(
        body,
        out_shape=jax.ShapeDtypeStruct((TOK_PER_DEV, D_OUT), jnp.float32),
        in_specs=[
            pl.BlockSpec(memory_space=pltpu.VMEM),
            pl.BlockSpec(memory_space=pltpu.VMEM),
            pl.BlockSpec(memory_space=pltpu.VMEM),
        ],
        out_specs=pl.BlockSpec(memory_space=pltpu.VMEM),
        scratch_shapes=[
            pltpu.VMEM((N_TOK, 1), jnp.float32),
            pltpu.VMEM((N_TOK, 1), jnp.int32),
            pltpu.VMEM((N_SLOT, D_OUT), jnp.float32),
            pltpu.SemaphoreType.DMA((N_SLOT,)),
            pltpu.SemaphoreType.DMA((TOK_PER_DEV,)),
        ],
        compiler_params=pltpu.CompilerParams(collective_id=0),
    )(x, route_idx.reshape(1, N_TOK), expert_W)


# device time: 9899 ns/iter; 1.1867x vs baseline; 1.1867x over previous
import jax
import jax.numpy as jnp
from jax import lax
from jax.experimental import pallas as pl
from jax.experimental.pallas import tpu as pltpu

N_DEV = 16
N_TOK = 256
D_IN = 128
D_OUT = 256
N_EXP = 32
EXP_PER_DEV = N_EXP // N_DEV
CAP = 6
N_SLOT = EXP_PER_DEV * CAP
TOK_PER_DEV = N_TOK // N_DEV


def kernel(x, router_W, route_idx, expert_W):
    def body(x_hbm, idx_hbm, w_hbm, out_ref,
             x_ref, idx_ref, w_ref, meta_ref, vcol_ref, send_buf,
             send_sems, recv_sems, in_sems):
        p = lax.axis_index("i")

        xcp = pltpu.make_async_copy(x_hbm, x_ref, in_sems.at[0])
        icp = pltpu.make_async_copy(idx_hbm, idx_ref, in_sems.at[1])
        wcp = pltpu.make_async_copy(w_hbm, w_ref, in_sems.at[2])
        xcp.start()
        icp.start()
        wcp.start()

        out_ref[...] = jnp.zeros_like(out_ref)

        barrier = pltpu.get_barrier_semaphore()
        for d in range(1, N_DEV):
            pl.semaphore_signal(
                barrier, inc=1,
                device_id=((p + d) % N_DEV,),
                device_id_type=pl.DeviceIdType.MESH,
            )

        icp.wait()
        v = idx_ref[...].reshape(N_TOK, 1)
        vcol_ref[...] = v
        lane_e = lax.broadcasted_iota(jnp.int32, (N_TOK, N_EXP), 1)
        one_hot = (v == lane_e).astype(jnp.float32)
        row_i = lax.broadcasted_iota(jnp.int32, (N_TOK, N_TOK), 0)
        col_i = lax.broadcasted_iota(jnp.int32, (N_TOK, N_TOK), 1)
        lower = (col_i <= row_i).astype(jnp.float32)
        counts = jnp.dot(lower.astype(jnp.bfloat16),
                         one_hot.astype(jnp.bfloat16),
                         preferred_element_type=jnp.float32)
        occ = jnp.sum(one_hot * counts, axis=1, keepdims=True)
        meta_ref[...] = occ

        s_iota = lax.broadcasted_iota(jnp.int32, (N_TOK, N_SLOT), 1)
        e_s = p * EXP_PER_DEV + s_iota // CAP
        c_s = (s_iota % CAP + 1).astype(jnp.float32)
        S = jnp.logical_and(v == e_s, occ == c_s).astype(jnp.float32)

        cdims = (((0,), (0,)), ((), ()))
        xcp.wait()
        X = lax.dot_general(S, x_ref[...], cdims,
                            preferred_element_type=jnp.float32)
        row_iota = lax.broadcasted_iota(
            jnp.int32, (N_TOK, 2), 0).astype(jnp.float32)
        aux = jnp.where(
            lax.broadcasted_iota(jnp.int32, (N_TOK, 2), 1) == 0,
            row_iota, 1.0)
        R = lax.dot_general(S, aux, cdims,
                            preferred_element_type=jnp.float32)
        t_enc = R[:, :1] + 512.0 * (1.0 - jnp.minimum(R[:, 1:2], 1.0))

        X_bf = X.astype(jnp.bfloat16)
        wcp.wait()
        y0 = jnp.dot(X_bf, w_ref[0].astype(jnp.bfloat16),
                     preferred_element_type=jnp.float32)
        y1 = jnp.dot(X_bf, w_ref[1].astype(jnp.bfloat16),
                     preferred_element_type=jnp.float32)
        first_half = lax.broadcasted_iota(jnp.int32, (N_SLOT, 1), 0) < CAP
        send_buf[...] = jnp.where(first_half, y0, y1)

        occ_mine = meta_ref[pl.ds(p * TOK_PER_DEV, TOK_PER_DEV), :]
        v_mine = vcol_ref[pl.ds(p * TOK_PER_DEV, TOK_PER_DEV), :]
        expected = jnp.logical_and(
            occ_mine <= float(CAP), v_mine // EXP_PER_DEV != p
        ).astype(jnp.int32)
        j_iota = lax.broadcasted_iota(jnp.int32, (TOK_PER_DEV, 1), 0)
        exp_bits = jnp.sum(expected << j_iota)

        slot_enc = [t_enc[s, 0].astype(jnp.int32) for s in range(N_SLOT)]
        slot_on = [t < N_TOK for t in slot_enc]
        slot_dst = [t % N_TOK // TOK_PER_DEV for t in slot_enc]
        slot_row = [t % TOK_PER_DEV for t in slot_enc]

        pl.semaphore_wait(barrier, N_DEV - 1)

        def send_rdma(s):
            return pltpu.make_async_remote_copy(
                src_ref=send_buf.at[pl.ds(s, 1), :],
                dst_ref=out_ref.at[pl.ds(slot_row[s], 1), :],
                send_sem=send_sems.at[s],
                recv_sem=recv_sems.at[slot_row[s]],
                device_id=(slot_dst[s],),
                device_id_type=pl.DeviceIdType.MESH,
            )

        for s in range(N_SLOT):
            @pl.when(jnp.logical_and(slot_on[s], slot_dst[s] == p))
            def _(s=s):
                out_ref[pl.ds(slot_row[s], 1), :] = send_buf[pl.ds(s, 1), :]

            @pl.when(jnp.logical_and(slot_on[s], slot_dst[s] != p))
            def _(s=s):
                send_rdma(s).start()

        for j in range(TOK_PER_DEV):
            @pl.when(jnp.bitwise_and(lax.shift_right_logical(exp_bits, j), 1) == 1)
            def _(j=j):
                recv = pltpu.make_async_remote_copy(
                    src_ref=send_buf.at[pl.ds(0, 1), :],
                    dst_ref=out_ref.at[pl.ds(j, 1), :],
                    send_sem=send_sems.at[0],
                    recv_sem=recv_sems.at[j],
                    device_id=(0,),
                    device_id_type=pl.DeviceIdType.MESH,
                )
                recv.wait_recv()

        for s in range(N_SLOT):
            @pl.when(jnp.logical_and(slot_on[s], slot_dst[s] != p))
            def _(s=s):
                send_rdma(s).wait_send()

    hbm = pltpu.MemorySpace.HBM
    x = pltpu.with_memory_space_constraint(x, hbm)
    idx2 = pltpu.with_memory_space_constraint(route_idx.reshape(1, N_TOK), hbm)
    expert_W = pltpu.with_memory_space_constraint(expert_W, hbm)
    return pl.pallas_call(
        body,
        out_shape=jax.ShapeDtypeStruct((TOK_PER_DEV, D_OUT), jnp.float32),
        in_specs=[
            pl.BlockSpec(memory_space=hbm),
            pl.BlockSpec(memory_space=hbm),
            pl.BlockSpec(memory_space=hbm),
        ],
        out_specs=pl.BlockSpec(memory_space=pltpu.VMEM),
        scratch_shapes=[
            pltpu.VMEM((N_TOK, D_IN), jnp.float32),
            pltpu.VMEM((1, N_TOK), jnp.int32),
            pltpu.VMEM((EXP_PER_DEV, D_IN, D_OUT), jnp.float32),
            pltpu.VMEM((N_TOK, 1), jnp.float32),
            pltpu.VMEM((N_TOK, 1), jnp.int32),
            pltpu.VMEM((N_SLOT, D_OUT), jnp.float32),
            pltpu.SemaphoreType.DMA((N_SLOT,)),
            pltpu.SemaphoreType.DMA((TOK_PER_DEV,)),
            pltpu.SemaphoreType.DMA((3,)),
        ],
        compiler_params=pltpu.CompilerParams(collective_id=0),
    )(x, idx2, expert_W)
